# baseline (speedup 1.0000x reference)
import jax
import jax.numpy as jnp
from jax.experimental import pallas as pl
from jax.experimental.pallas import tpu as pltpu

_B = 2
_S = 2048
_H = 2048
_E = 8
_A = 128
_T = _B * _S
_TILE = 512


def _dot_nt(a, b):
    return jax.lax.dot_general(
        a, b, (((1,), (1,)), ((), ())), preferred_element_type=jnp.float32)


def _fused_body(xi_ref, xo_ref, xr_ref, wr_ref, wd_ref, wu_ref, exp_ref,
                out_ref, lg_ref):
    xr = xr_ref[...]
    xr_hi = xr.astype(jnp.bfloat16)
    xr_lo = (xr - xr_hi.astype(jnp.float32)).astype(jnp.bfloat16)
    wr = wr_ref[...]
    wr_hi = wr.astype(jnp.bfloat16)
    wr_lo = (wr - wr_hi.astype(jnp.float32)).astype(jnp.bfloat16)
    logits = (_dot_nt(xr_hi, wr_hi)
              + (_dot_nt(xr_lo, wr_hi) + _dot_nt(xr_hi, wr_lo)))
    lg_ref[...] = logits

    cols = jax.lax.broadcasted_iota(jnp.int32, logits.shape, 1)
    l1 = jnp.max(logits, axis=-1, keepdims=True)
    i1 = jnp.argmax(logits, axis=-1, keepdims=True)
    m1 = cols == i1
    l_rest = jnp.where(m1, -jnp.inf, logits)
    l2 = jnp.max(l_rest, axis=-1, keepdims=True)
    i2 = jnp.argmax(l_rest, axis=-1, keepdims=True)
    m2 = cols == i2
    e2 = jnp.exp(l2 - l1)
    w1 = 1.0 / (1.0 + e2)
    w = jnp.where(m1, w1, jnp.where(m2, e2 * w1, 0.0))

    xi = xi_ref[...].astype(jnp.bfloat16)
    wd = wd_ref[...].astype(jnp.bfloat16)
    h = _dot_nt(xi, wd)
    h = jax.nn.gelu(h.astype(jnp.bfloat16))
    w_exp = jax.lax.dot_general(
        w.astype(jnp.bfloat16), exp_ref[...], (((1,), (0,)), ((), ())),
        preferred_element_type=jnp.float32).astype(jnp.bfloat16)
    hw = h * w_exp

    acc = xo_ref[...]
    for e in range(_E):
        wu_e = wu_ref[e].astype(jnp.bfloat16)
        acc = acc + _dot_nt(hw[:, e * _A:(e + 1) * _A], wu_e)
    out_ref[...] = acc


@jax.jit
def kernel(input_hidden_states, output_hidden_states, router_hidden_states,
           W_router, W_down, W_up):
    orig_shape = output_hidden_states.shape
    xi = input_hidden_states.reshape(_T, _H)
    xo = output_hidden_states.reshape(_T, _H)
    xr = router_hidden_states.reshape(_T, _H)
    wd = W_down.reshape(_E * _A, _H)
    expand = jnp.repeat(jnp.eye(_E, dtype=jnp.bfloat16), _A, axis=1)

    grid = (_T // _TILE,)
    out, logits = pl.pallas_call(
        _fused_body,
        grid=grid,
        in_specs=[
            pl.BlockSpec((_TILE, _H), lambda i: (i, 0)),
            pl.BlockSpec((_TILE, _H), lambda i: (i, 0)),
            pl.BlockSpec((_TILE, _H), lambda i: (i, 0)),
            pl.BlockSpec((_E, _H), lambda i: (0, 0)),
            pl.BlockSpec((_E * _A, _H), lambda i: (0, 0)),
            pl.BlockSpec((_E, _H, _A), lambda i: (0, 0, 0)),
            pl.BlockSpec((_E, _E * _A), lambda i: (0, 0)),
        ],
        out_specs=[
            pl.BlockSpec((_TILE, _H), lambda i: (i, 0)),
            pl.BlockSpec((_TILE, _E), lambda i: (i, 0)),
        ],
        out_shape=[
            jax.ShapeDtypeStruct((_T, _H), jnp.float32),
            jax.ShapeDtypeStruct((_T, _E), jnp.float32),
        ],
        compiler_params=pltpu.CompilerParams(
            dimension_semantics=("arbitrary",),
        ),
    )(xi, xo, xr, W_router, wd, W_up, expand)

    return out.reshape(orig_shape), logits


# speedup vs baseline: 1.3952x; 1.3952x over previous
import jax
import jax.numpy as jnp
from jax.experimental import pallas as pl
from jax.experimental.pallas import tpu as pltpu

_B = 2
_S = 2048
_H = 2048
_E = 8
_A = 128
_T = _B * _S
_TILE = 256


def _dot_nn(a, b):
    return jax.lax.dot_general(
        a, b, (((1,), (0,)), ((), ())), preferred_element_type=jnp.float32)


def _fused_body(xi_ref, xo_ref, xr_ref, wrh_ref, wrl_ref, wd_ref, wu_ref,
                exp_ref, out_ref, lg_ref, wd_s, wu_s):
    step = pl.program_id(0)

    @pl.when(step == 0)
    def _pack_weights():
        wd_s[...] = wd_ref[...].T.astype(jnp.bfloat16)
        for e in range(_E):
            wu_s[e * _A:(e + 1) * _A, :] = wu_ref[e].T.astype(jnp.bfloat16)

    @pl.when(step > 0)
    def _compute_tile():
        xr = xr_ref[...]
        xr_hi = xr.astype(jnp.bfloat16)
        xr_lo = (xr - xr_hi.astype(jnp.float32)).astype(jnp.bfloat16)
        logits = (_dot_nn(xr_hi, wrh_ref[...])
                  + (_dot_nn(xr_lo, wrh_ref[...])
                     + _dot_nn(xr_hi, wrl_ref[...])))
        lg_ref[...] = logits

        cols = jax.lax.broadcasted_iota(jnp.int32, logits.shape, 1)
        l1 = jnp.max(logits, axis=-1, keepdims=True)
        i1 = jnp.argmax(logits, axis=-1, keepdims=True)
        m1 = cols == i1
        l_rest = jnp.where(m1, -jnp.inf, logits)
        l2 = jnp.max(l_rest, axis=-1, keepdims=True)
        i2 = jnp.argmax(l_rest, axis=-1, keepdims=True)
        m2 = cols == i2
        e2 = jnp.exp(l2 - l1)
        w1 = 1.0 / (1.0 + e2)
        w = jnp.where(m1, w1, jnp.where(m2, e2 * w1, 0.0))

        xi = xi_ref[...].astype(jnp.bfloat16)
        h = _dot_nn(xi, wd_s[...])
        h = jax.nn.gelu(h.astype(jnp.bfloat16))
        w_exp = _dot_nn(w.astype(jnp.bfloat16),
                        exp_ref[...]).astype(jnp.bfloat16)
        hw = h * w_exp
        res = _dot_nn(hw, wu_s[...])
        out_ref[...] = xo_ref[...] + res


@jax.jit
def kernel(input_hidden_states, output_hidden_states, router_hidden_states,
           W_router, W_down, W_up):
    orig_shape = output_hidden_states.shape
    xi = input_hidden_states.reshape(_T, _H)
    xo = output_hidden_states.reshape(_T, _H)
    xr = router_hidden_states.reshape(_T, _H)
    wr = W_router.T
    wr_hi = wr.astype(jnp.bfloat16)
    wr_lo = (wr - wr_hi.astype(jnp.float32)).astype(jnp.bfloat16)
    wd = W_down.reshape(_E * _A, _H)
    expand = jnp.repeat(jnp.eye(_E, dtype=jnp.bfloat16), _A, axis=1)

    def _tile(i):
        return jnp.where(i == 0, 0, i - 1)

    grid = (_T // _TILE + 1,)
    out, logits = pl.pallas_call(
        _fused_body,
        grid=grid,
        in_specs=[
            pl.BlockSpec((_TILE, _H), lambda i: (_tile(i), 0)),
            pl.BlockSpec((_TILE, _H), lambda i: (_tile(i), 0)),
            pl.BlockSpec((_TILE, _H), lambda i: (_tile(i), 0)),
            pl.BlockSpec((_H, _E), lambda i: (0, 0)),
            pl.BlockSpec((_H, _E), lambda i: (0, 0)),
            pl.BlockSpec((_E * _A, _H), lambda i: (0, 0)),
            pl.BlockSpec((_E, _H, _A), lambda i: (0, 0, 0)),
            pl.BlockSpec((_E, _E * _A), lambda i: (0, 0)),
        ],
        out_specs=[
            pl.BlockSpec((_TILE, _H), lambda i: (_tile(i), 0)),
            pl.BlockSpec((_TILE, _E), lambda i: (_tile(i), 0)),
        ],
        out_shape=[
            jax.ShapeDtypeStruct((_T, _H), jnp.float32),
            jax.ShapeDtypeStruct((_T, _E), jnp.float32),
        ],
        scratch_shapes=[
            pltpu.VMEM((_H, _E * _A), jnp.bfloat16),
            pltpu.VMEM((_E * _A, _H), jnp.bfloat16),
        ],
        compiler_params=pltpu.CompilerParams(
            dimension_semantics=("arbitrary",),
        ),
    )(xi, xo, xr, wr_hi, wr_lo, wd, W_up, expand)

    return out.reshape(orig_shape), logits
